# Initial kernel scaffold; baseline (speedup 1.0000x reference)
import jax
import jax.numpy as jnp
from jax.experimental import pallas as pl


def kernel(x):
    raise NotImplementedError("write your pallas kernel here")


# speedup vs baseline: 1.1943x; 1.1943x over previous
import jax
import jax.numpy as jnp
from jax.experimental import pallas as pl
from jax.experimental.pallas import tpu as pltpu

_LANES = 128


_SPLIT = 16


def _partial_kernel(x_ref, mx_ref, sx_ref):
    v = x_ref[0]
    v3 = v.reshape(_SPLIT, v.shape[0] // _SPLIT, _LANES)
    m3 = jnp.max(v3, axis=1)
    m = jnp.max(m3, axis=0, keepdims=True)
    s3 = jnp.sum(jnp.exp(v3 - m[None]), axis=1)
    s = jnp.sum(s3, axis=0, keepdims=True)
    mx_ref[0] = m
    sx_ref[0] = s


def _normalize_kernel(x_ref, mx_ref, sx_ref, o_ref):
    mp = mx_ref[:, 0, :]
    sp = sx_ref[:, 0, :]
    m_gl = jnp.max(jnp.max(mp, axis=0, keepdims=True), axis=1, keepdims=True)
    s_gl = jnp.sum(
        jnp.sum(sp * jnp.exp(mp - m_gl), axis=0, keepdims=True),
        axis=1, keepdims=True)
    r = 1.0 / s_gl
    o_ref[0] = jnp.exp(x_ref[0] - m_gl) * r


def _softmax_pallas(x, num_chunks):
    n = x.shape[0]
    rows = n // _LANES
    r_rows = rows // num_chunks
    x3 = x.reshape(num_chunks, r_rows, _LANES)

    part_shape = jax.ShapeDtypeStruct((num_chunks, 1, _LANES), jnp.float32)
    mx, sx = pl.pallas_call(
        _partial_kernel,
        out_shape=(part_shape, part_shape),
        grid=(num_chunks,),
        in_specs=[pl.BlockSpec((1, r_rows, _LANES), lambda i: (i, 0, 0))],
        out_specs=(pl.BlockSpec((1, 1, _LANES), lambda i: (i, 0, 0)),
                   pl.BlockSpec((1, 1, _LANES), lambda i: (i, 0, 0))),
        compiler_params=pltpu.CompilerParams(
            dimension_semantics=("parallel",)),
        name="softmax_partials",
    )(x3)

    out3 = pl.pallas_call(
        _normalize_kernel,
        out_shape=jax.ShapeDtypeStruct((num_chunks, r_rows, _LANES), jnp.float32),
        grid=(num_chunks,),
        in_specs=[
            pl.BlockSpec((1, r_rows, _LANES), lambda i: (i, 0, 0)),
            pl.BlockSpec((num_chunks, 1, _LANES), lambda i: (0, 0, 0)),
            pl.BlockSpec((num_chunks, 1, _LANES), lambda i: (0, 0, 0)),
        ],
        out_specs=pl.BlockSpec((1, r_rows, _LANES), lambda i: (i, 0, 0)),
        compiler_params=pltpu.CompilerParams(
            dimension_semantics=("parallel",)),
        name="softmax_normalize",
    )(x3, mx, sx)

    return out3.reshape(n)


def kernel(x):
    return _softmax_pallas(x, num_chunks=32)


# speedup vs baseline: 1.3098x; 1.0967x over previous
import jax
import jax.numpy as jnp
from jax.experimental import pallas as pl
from jax.experimental.pallas import tpu as pltpu

_LANES = 128


_SPLIT = 16


def _partial_kernel(x_ref, mx_ref, sx_ref):
    v = x_ref[0]
    v3 = v.reshape(_SPLIT, v.shape[0] // _SPLIT, _LANES)
    m3 = jnp.max(v3, axis=1)
    m = jnp.max(m3, axis=0, keepdims=True)
    s3 = jnp.sum(jnp.exp(v3 - m[None]), axis=1)
    s = jnp.sum(s3, axis=0, keepdims=True)
    mx_ref[0] = m
    sx_ref[0] = s


def _normalize_kernel(x_ref, mx_ref, sx_ref, o_ref):
    mp = mx_ref[:, 0, :]
    sp = sx_ref[:, 0, :]
    m_gl = jnp.max(jnp.max(mp, axis=0, keepdims=True), axis=1, keepdims=True)
    s_gl = jnp.sum(
        jnp.sum(sp * jnp.exp(mp - m_gl), axis=0, keepdims=True),
        axis=1, keepdims=True)
    r = 1.0 / s_gl
    o_ref[0] = jnp.exp(x_ref[0] - m_gl) * r


def _softmax_pallas(x, c1, c2):
    n = x.shape[0]
    rows = n // _LANES
    r1 = rows // c1
    r2 = rows // c2
    x1 = x.reshape(c1, r1, _LANES)
    x2 = x.reshape(c2, r2, _LANES)

    part_shape = jax.ShapeDtypeStruct((c1, 1, _LANES), jnp.float32)
    mx, sx = pl.pallas_call(
        _partial_kernel,
        out_shape=(part_shape, part_shape),
        grid=(c1,),
        in_specs=[pl.BlockSpec((1, r1, _LANES), lambda i: (i, 0, 0))],
        out_specs=(pl.BlockSpec((1, 1, _LANES), lambda i: (i, 0, 0)),
                   pl.BlockSpec((1, 1, _LANES), lambda i: (i, 0, 0))),
        compiler_params=pltpu.CompilerParams(
            dimension_semantics=("parallel",),
            vmem_limit_bytes=56 * 1024 * 1024),
        name="softmax_partials",
    )(x1)

    out3 = pl.pallas_call(
        _normalize_kernel,
        out_shape=jax.ShapeDtypeStruct((c2, r2, _LANES), jnp.float32),
        grid=(c2,),
        in_specs=[
            pl.BlockSpec((1, r2, _LANES), lambda i: (i, 0, 0)),
            pl.BlockSpec((c1, 1, _LANES), lambda i: (0, 0, 0)),
            pl.BlockSpec((c1, 1, _LANES), lambda i: (0, 0, 0)),
        ],
        out_specs=pl.BlockSpec((1, r2, _LANES), lambda i: (i, 0, 0)),
        compiler_params=pltpu.CompilerParams(
            dimension_semantics=("parallel",),
            vmem_limit_bytes=56 * 1024 * 1024),
        name="softmax_normalize",
    )(x2, mx, sx)

    return out3.reshape(n)


def kernel(x):
    return _softmax_pallas(x, c1=8, c2=16)
